# baseline (speedup 1.0000x reference)
import functools

import jax
import jax.numpy as jnp
from jax import lax
from jax.experimental import pallas as pl
from jax.experimental.pallas import tpu as pltpu
from jax.experimental.pallas import tpu_sc as plsc

N = 10000
D_IN = 128
D_OUT = 200
E = 320000

NC = 2
NS = 16
NW = NC * NS
CHK = 80
CPT = 125
D_PAD = 256
R_STEP = 624
R_LEN = 640
N1 = 10240
R1D = N1 // NS
HBLK = 64

_mesh = plsc.VectorSubcoreMesh(core_axis_name="c", subcore_axis_name="s")


@functools.partial(
    pl.kernel, mesh=_mesh,
    out_type=jax.ShapeDtypeStruct((NC * N1,), jnp.float32),
    scratch_types=[
        pltpu.VMEM((CPT, CHK), jnp.int32),
        pltpu.VMEM((CHK,), jnp.float32),
        pltpu.VMEM_SHARED((N1,), jnp.float32),
    ],
)
def _hist_k(ei3_hbm, ones_hbm, zeros1_hbm, deg_hbm, dst_v, ones_v, deg_sh):
    cid = lax.axis_index("c")
    sid = lax.axis_index("s")
    wid = sid * NC + cid
    r0 = sid * R1D
    pltpu.sync_copy(zeros1_hbm.at[pl.ds(r0, R1D)], deg_sh.at[pl.ds(r0, R1D)])
    pltpu.sync_copy(ei3_hbm.at[NW + wid], dst_v)
    pltpu.sync_copy(ones_hbm, ones_v)
    plsc.subcore_barrier()

    def body(j, carry):
        pltpu.sync_copy(ones_v, deg_sh.at[dst_v.at[j]], add=True)
        return carry

    lax.fori_loop(0, CPT, body, 0)
    plsc.subcore_barrier()
    pltpu.sync_copy(deg_sh.at[pl.ds(r0, R1D)],
                    deg_hbm.at[pl.ds(cid * N1 + r0, R1D)])


@functools.partial(
    pl.kernel, mesh=_mesh,
    out_type=jax.ShapeDtypeStruct((NC * N1, D_IN), jnp.float32),
    scratch_types=[
        pltpu.VMEM((HBLK, CHK), jnp.int32),
        pltpu.VMEM((HBLK, CHK), jnp.int32),
        pltpu.VMEM((CHK, D_IN), jnp.float32),
        pltpu.VMEM((CHK, D_IN), jnp.float32),
        pltpu.VMEM_SHARED((N, D_IN), jnp.float32),
        pltpu.SemaphoreType.DMA,
        pltpu.SemaphoreType.DMA,
    ],
)
def _scatter_k(ei3_hbm, g_hbm, zeros_hbm, out_hbm,
               src_v, dst_v, rows_a, rows_b, acc_sh, sem_a, sem_b):
    cid = lax.axis_index("c")
    sid = lax.axis_index("s")
    wid = sid * NC + cid
    r0 = sid * R_STEP

    @pl.when(cid == 0)
    def _():
        pltpu.sync_copy(g_hbm.at[pl.ds(r0, R_LEN)], acc_sh.at[pl.ds(r0, R_LEN)])

    @pl.when(cid != 0)
    def _():
        pltpu.sync_copy(zeros_hbm.at[pl.ds(r0, R_LEN)], acc_sh.at[pl.ds(r0, R_LEN)])

    plsc.subcore_barrier()

    def fire(j, rows, sem):
        pltpu.async_copy(g_hbm.at[src_v.at[j]], rows, sem)

    def drain(rows, sem):
        pltpu.make_async_copy(g_hbm.at[pl.ds(0, CHK)], rows, sem).wait()

    def scat(j, rows):
        pltpu.sync_copy(rows, acc_sh.at[dst_v.at[j]], add=True)

    def run_half(start, L):
        if L == HBLK:
            pltpu.sync_copy(ei3_hbm.at[wid, pl.ds(start, HBLK)], src_v)
            pltpu.sync_copy(ei3_hbm.at[NW + wid, pl.ds(start, HBLK)], dst_v)
        else:
            pltpu.sync_copy(ei3_hbm.at[wid, pl.ds(start, L)],
                            src_v.at[pl.ds(0, L)])
            pltpu.sync_copy(ei3_hbm.at[NW + wid, pl.ds(start, L)],
                            dst_v.at[pl.ds(0, L)])
        fire(0, rows_a, sem_a)
        fire(1, rows_b, sem_b)
        if L % 2 == 0:
            def body(k, carry):
                j = 2 * k
                drain(rows_a, sem_a)
                scat(j, rows_a)
                fire(j + 2, rows_a, sem_a)
                drain(rows_b, sem_b)
                scat(j + 1, rows_b)
                fire(j + 3, rows_b, sem_b)
                return carry
            lax.fori_loop(0, (L - 2) // 2, body, 0)
            drain(rows_a, sem_a)
            scat(L - 2, rows_a)
            drain(rows_b, sem_b)
            scat(L - 1, rows_b)
        else:
            def body(k, carry):
                j = 2 * k
                drain(rows_a, sem_a)
                scat(j, rows_a)
                fire(j + 2, rows_a, sem_a)
                drain(rows_b, sem_b)
                scat(j + 1, rows_b)
                fire(j + 3, rows_b, sem_b)
                return carry
            lax.fori_loop(0, (L - 3) // 2, body, 0)
            drain(rows_a, sem_a)
            scat(L - 3, rows_a)
            fire(L - 1, rows_a, sem_a)
            drain(rows_b, sem_b)
            scat(L - 2, rows_b)
            drain(rows_a, sem_a)
            scat(L - 1, rows_a)

    run_half(0, HBLK)
    run_half(HBLK, CPT - HBLK)

    plsc.subcore_barrier()
    pltpu.sync_copy(acc_sh.at[pl.ds(r0, R_LEN)],
                    out_hbm.at[pl.ds(cid * N1 + r0, R_LEN)])


BRS = 2048


def _expand_rows(v):
    v3 = lax.broadcast_in_dim(v, (BRS // 128, 128, 128), (0, 1))
    return v3.reshape(BRS, 128)


def _scale_body(p_ref, x_ref, g_ref, dinv_ref):
    deg = p_ref[0] + p_ref[1] + 1.0
    dinv = lax.rsqrt(deg)
    dinv_ref[...] = dinv
    g_ref[...] = _expand_rows(dinv) * x_ref[...]


def _mm_body(q0_ref, q1_ref, dinv_ref, w1_ref, b1_ref, w2_ref, b2_ref,
             mu_ref, ls_ref):
    a = (q0_ref[...] + q1_ref[...]) * _expand_rows(dinv_ref[...])
    mu_ref[...] = jnp.dot(a, w1_ref[...], preferred_element_type=jnp.float32) + b1_ref[...]
    ls_ref[...] = jnp.dot(a, w2_ref[...], preferred_element_type=jnp.float32) + b2_ref[...]


_NB = N // 1000


def kernel(x, edge_index, W1, b1, W2, b2):
    ei3 = edge_index.reshape(2 * NW, CPT, CHK)

    ones = jnp.ones((CHK,), jnp.float32)
    zeros1 = jnp.zeros((N1,), jnp.float32)
    zeros2 = jnp.zeros((N, D_IN), jnp.float32)

    deg_p = _hist_k(ei3, ones, zeros1)
    deg3 = deg_p.reshape(2, N1 // 128, 128)

    g, dinv = pl.pallas_call(
        _scale_body,
        grid=(N1 // BRS,),
        in_specs=[
            pl.BlockSpec((2, BRS // 128, 128), lambda i: (0, i, 0)),
            pl.BlockSpec((BRS, D_IN), lambda i: (i, 0)),
        ],
        out_specs=[
            pl.BlockSpec((BRS, D_IN), lambda i: (i, 0)),
            pl.BlockSpec((BRS // 128, 128), lambda i: (i, 0)),
        ],
        out_shape=[
            jax.ShapeDtypeStruct((N1, D_IN), jnp.float32),
            jax.ShapeDtypeStruct((N1 // 128, 128), jnp.float32),
        ],
    )(deg3, x)

    parts = _scatter_k(ei3, g, zeros2)

    mu, ls = pl.pallas_call(
        _mm_body,
        grid=(N1 // BRS,),
        in_specs=[
            pl.BlockSpec((BRS, D_IN), lambda i: (i, 0)),
            pl.BlockSpec((BRS, D_IN), lambda i: (i + N1 // BRS, 0)),
            pl.BlockSpec((BRS // 128, 128), lambda i: (i, 0)),
            pl.BlockSpec((D_IN, D_OUT), lambda i: (0, 0)),
            pl.BlockSpec((1, D_OUT), lambda i: (0, 0)),
            pl.BlockSpec((D_IN, D_OUT), lambda i: (0, 0)),
            pl.BlockSpec((1, D_OUT), lambda i: (0, 0)),
        ],
        out_specs=[
            pl.BlockSpec((BRS, D_OUT), lambda i: (i, 0)),
            pl.BlockSpec((BRS, D_OUT), lambda i: (i, 0)),
        ],
        out_shape=[
            jax.ShapeDtypeStruct((N1, D_OUT), jnp.float32),
            jax.ShapeDtypeStruct((N1, D_OUT), jnp.float32),
        ],
    )(parts, parts, dinv, W1, b1.reshape(1, D_OUT), W2, b2.reshape(1, D_OUT))

    return (mu[:N], ls[:N])


# speedup vs baseline: 1.3571x; 1.3571x over previous
import functools

import jax
import jax.numpy as jnp
from jax import lax
from jax.experimental import pallas as pl
from jax.experimental.pallas import tpu as pltpu
from jax.experimental.pallas import tpu_sc as plsc

N = 10000
D_IN = 128
D_OUT = 200
E = 320000

NC = 2
NS = 16
NW = NC * NS
CHK = 80
CPT = 125
D_PAD = 256
R_STEP = 624
R_LEN = 640
N1 = 10240
R1D = N1 // NS
HBLK = 32

_mesh = plsc.VectorSubcoreMesh(core_axis_name="c", subcore_axis_name="s")


@functools.partial(
    pl.kernel, mesh=_mesh,
    out_type=jax.ShapeDtypeStruct((NC * N1,), jnp.float32),
    scratch_types=[
        pltpu.VMEM((CPT, CHK), jnp.int32),
        pltpu.VMEM((CHK,), jnp.float32),
        pltpu.VMEM_SHARED((N1,), jnp.float32),
    ],
)
def _hist_k(ei3_hbm, ones_hbm, zeros1_hbm, deg_hbm, dst_v, ones_v, deg_sh):
    cid = lax.axis_index("c")
    sid = lax.axis_index("s")
    wid = sid * NC + cid
    r0 = sid * R1D
    pltpu.sync_copy(zeros1_hbm.at[pl.ds(r0, R1D)], deg_sh.at[pl.ds(r0, R1D)])
    pltpu.sync_copy(ei3_hbm.at[NW + wid], dst_v)
    pltpu.sync_copy(ones_hbm, ones_v)
    plsc.subcore_barrier()

    def body(j, carry):
        pltpu.sync_copy(ones_v, deg_sh.at[dst_v.at[j]], add=True)
        return carry

    lax.fori_loop(0, CPT, body, 0)
    plsc.subcore_barrier()
    pltpu.sync_copy(deg_sh.at[pl.ds(r0, R1D)],
                    deg_hbm.at[pl.ds(cid * N1 + r0, R1D)])


@functools.partial(
    pl.kernel, mesh=_mesh,
    out_type=jax.ShapeDtypeStruct((NC * N, D_IN), jnp.float32),
    scratch_types=[
        pltpu.VMEM((HBLK, CHK), jnp.int32),
        pltpu.VMEM((HBLK, CHK), jnp.int32),
        pltpu.VMEM((CHK, D_IN), jnp.float32),
        pltpu.VMEM((CHK, D_IN), jnp.float32),
        pltpu.VMEM((CHK, D_IN), jnp.float32),
        pltpu.VMEM_SHARED((N, D_IN), jnp.float32),
        pltpu.SemaphoreType.DMA,
        pltpu.SemaphoreType.DMA,
        pltpu.SemaphoreType.DMA,
    ],
)
def _scatter_k(ei3_hbm, g_hbm, zeros_hbm, out_hbm,
               src_v, dst_v, rows_a, rows_b, rows_c, acc_sh,
               sem_a, sem_b, sem_c):
    cid = lax.axis_index("c")
    sid = lax.axis_index("s")
    wid = sid * NC + cid
    r0 = sid * R_STEP

    @pl.when(cid == 0)
    def _():
        pltpu.sync_copy(g_hbm.at[pl.ds(r0, R_LEN)], acc_sh.at[pl.ds(r0, R_LEN)])

    @pl.when(cid != 0)
    def _():
        pltpu.sync_copy(zeros_hbm.at[pl.ds(r0, R_LEN)], acc_sh.at[pl.ds(r0, R_LEN)])

    plsc.subcore_barrier()

    def fire(j, rows, sem):
        pltpu.async_copy(g_hbm.at[src_v.at[j]], rows, sem)

    def drain(rows, sem):
        pltpu.make_async_copy(g_hbm.at[pl.ds(0, CHK)], rows, sem).wait()

    def scat(j, rows):
        pltpu.sync_copy(rows, acc_sh.at[dst_v.at[j]], add=True)

    bufs = ((rows_a, sem_a), (rows_b, sem_b), (rows_c, sem_c))

    def run_block(start, L):
        if L == HBLK:
            pltpu.sync_copy(ei3_hbm.at[wid, pl.ds(start, HBLK)], src_v)
            pltpu.sync_copy(ei3_hbm.at[NW + wid, pl.ds(start, HBLK)], dst_v)
        else:
            pltpu.sync_copy(ei3_hbm.at[wid, pl.ds(start, L)],
                            src_v.at[pl.ds(0, L)])
            pltpu.sync_copy(ei3_hbm.at[NW + wid, pl.ds(start, L)],
                            dst_v.at[pl.ds(0, L)])
        for t in range(3):
            fire(t, *bufs[t])
        K = (L - 6) // 3 + 1

        def body(k, carry):
            j = 3 * k
            for t in range(3):
                drain(*bufs[t])
                scat(j + t, bufs[t][0])
                fire(j + t + 3, *bufs[t])
            return carry

        lax.fori_loop(0, K, body, 0)
        for j in range(3 * K, L):
            drain(*bufs[j % 3])
            scat(j, bufs[j % 3][0])
            if j + 3 <= L - 1:
                fire(j + 3, *bufs[j % 3])

    run_block(0, 32)
    run_block(32, 32)
    run_block(64, 32)
    run_block(96, 29)

    plsc.subcore_barrier()
    pltpu.sync_copy(acc_sh.at[pl.ds(r0, R_LEN)],
                    out_hbm.at[pl.ds(cid * N + r0, R_LEN)])


def _scale_body(p0_ref, p1_ref, x_ref, g_ref, dinv_ref):
    deg = p0_ref[...] + p1_ref[...] + 1.0
    dinv = lax.rsqrt(deg)
    dinv_ref[...] = dinv
    g_ref[...] = dinv * x_ref[...]


def _mm_body(q0_ref, q1_ref, dinv_ref, w1_ref, b1_ref, w2_ref, b2_ref,
             mu_ref, ls_ref):
    a = (q0_ref[...] + q1_ref[...]) * dinv_ref[...]
    mu_ref[...] = jnp.dot(a, w1_ref[...], preferred_element_type=jnp.float32) + b1_ref[...]
    ls_ref[...] = jnp.dot(a, w2_ref[...], preferred_element_type=jnp.float32) + b2_ref[...]


_NB = N // 1000


def kernel(x, edge_index, W1, b1, W2, b2):
    ei3 = edge_index.reshape(2 * NW, CPT, CHK)

    ones = jnp.ones((CHK,), jnp.float32)
    zeros1 = jnp.zeros((N1,), jnp.float32)
    zeros2 = jnp.zeros((N, D_IN), jnp.float32)

    deg_p = _hist_k(ei3, ones, zeros1)
    p0c = deg_p[:N].reshape(N, 1)
    p1c = deg_p[N1:N1 + N].reshape(N, 1)

    BR = 1000
    g, dinv = pl.pallas_call(
        _scale_body,
        grid=(N // BR,),
        in_specs=[
            pl.BlockSpec((BR, 1), lambda i: (i, 0)),
            pl.BlockSpec((BR, 1), lambda i: (i, 0)),
            pl.BlockSpec((BR, D_IN), lambda i: (i, 0)),
        ],
        out_specs=[
            pl.BlockSpec((BR, D_IN), lambda i: (i, 0)),
            pl.BlockSpec((BR, 1), lambda i: (i, 0)),
        ],
        out_shape=[
            jax.ShapeDtypeStruct((N, D_IN), jnp.float32),
            jax.ShapeDtypeStruct((N, 1), jnp.float32),
        ],
    )(p0c, p1c, x)

    parts = _scatter_k(ei3, g, zeros2)

    BRM = 2000
    mu, ls = pl.pallas_call(
        _mm_body,
        grid=(N // BRM,),
        in_specs=[
            pl.BlockSpec((BRM, D_IN), lambda i: (i, 0)),
            pl.BlockSpec((BRM, D_IN), lambda i: (i + N // 2000, 0)),
            pl.BlockSpec((BRM, 1), lambda i: (i, 0)),
            pl.BlockSpec((D_IN, D_OUT), lambda i: (0, 0)),
            pl.BlockSpec((1, D_OUT), lambda i: (0, 0)),
            pl.BlockSpec((D_IN, D_OUT), lambda i: (0, 0)),
            pl.BlockSpec((1, D_OUT), lambda i: (0, 0)),
        ],
        out_specs=[
            pl.BlockSpec((BRM, D_OUT), lambda i: (i, 0)),
            pl.BlockSpec((BRM, D_OUT), lambda i: (i, 0)),
        ],
        out_shape=[
            jax.ShapeDtypeStruct((N, D_OUT), jnp.float32),
            jax.ShapeDtypeStruct((N, D_OUT), jnp.float32),
        ],
    )(parts, parts, dinv, W1, b1.reshape(1, D_OUT), W2, b2.reshape(1, D_OUT))

    return (mu, ls)
